# baseline (speedup 1.0000x reference)
import functools

import jax
import jax.numpy as jnp
from jax.experimental import pallas as pl
from jax.experimental.pallas import tpu as pltpu

T = 2048
D = 1024
E = 64
F = 512
K = 2
BM = 64
P = 8192
NB = P // BM


CH = 128
NCH = T // CH


def _router_body(x_ref, gw_ref, d0_ref, d1_ref, w1_ref, w2_ref,
                 be_ref, used_ref):
    x = x_ref[...]
    gw = gw_ref[...]
    logits = jax.lax.dot_general(x, gw, (((1,), (1,)), ((), ())),
                                 preferred_element_type=jnp.float32)
    m1 = jnp.max(logits, axis=1, keepdims=True)
    iota = jax.lax.broadcasted_iota(jnp.int32, logits.shape, 1)
    big = jnp.int32(1 << 30)
    i1 = jnp.min(jnp.where(logits == m1, iota, big), axis=1, keepdims=True)
    mask1 = iota == i1
    m2 = jnp.max(jnp.where(mask1, -jnp.inf, logits), axis=1, keepdims=True)
    i2 = jnp.min(jnp.where((logits == m2) & (~mask1), iota, big),
                 axis=1, keepdims=True)
    w1 = 1.0 / (1.0 + jnp.exp(m2 - m1))
    w1_ref[...] = w1
    w2_ref[...] = 1.0 - w1

    oh1 = (iota == i1).astype(jnp.float32)
    oh2 = (iota == i2).astype(jnp.float32)
    ohc = oh1 + oh2
    ri = jax.lax.broadcasted_iota(jnp.int32, (CH, CH), 0)
    ci = jax.lax.broadcasted_iota(jnp.int32, (CH, CH), 1)
    tril = (ci < ri).astype(jnp.float32)
    base_rows = []
    running = jnp.zeros((1, E), jnp.float32)
    for c in range(NCH):
        chunk = ohc[c * CH:(c + 1) * CH]
        excl = jnp.dot(tril, chunk, preferred_element_type=jnp.float32)
        base_rows.append(excl + running)
        running = running + jnp.sum(chunk, axis=0, keepdims=True)
    base = jnp.concatenate(base_rows, axis=0)
    counts = running

    nblk = jnp.ceil(counts / BM)
    ei = jax.lax.broadcasted_iota(jnp.int32, (E, E), 0)
    ej = jax.lax.broadcasted_iota(jnp.int32, (E, E), 1)
    triu_inc = (ei <= ej).astype(jnp.float32)
    bend = jnp.dot(nblk, triu_inc, preferred_element_type=jnp.float32)
    pstart = (bend - nblk) * BM
    d0 = jnp.sum(oh1 * (pstart + base), axis=1, keepdims=True)
    d1 = jnp.sum(oh2 * (pstart + base), axis=1, keepdims=True)
    d0_ref[...] = d0.astype(jnp.int32)
    d1_ref[...] = d1.astype(jnp.int32)

    used2d = jax.lax.slice(bend, (0, E - 1), (1, E))
    used = used2d[0, 0]
    used_ref[...] = used2d.astype(jnp.int32)
    blki = jax.lax.broadcasted_iota(jnp.int32, (NB, E), 0).astype(jnp.float32)
    be_raw = jnp.sum((bend <= blki).astype(jnp.float32), axis=1, keepdims=True)
    eidx = jax.lax.broadcasted_iota(jnp.int32, (1, E), 1).astype(jnp.float32)
    last_e = jnp.max(jnp.where(counts > 0, eidx, -1.0))
    blk1 = jax.lax.broadcasted_iota(jnp.int32, (NB, 1), 0).astype(jnp.float32)
    be = jnp.where(blk1 < used, be_raw, last_e)
    be_ref[...] = be.astype(jnp.int32)


def _run_router(x2d, gate_w):
    out_shapes = (
        jax.ShapeDtypeStruct((T, 1), jnp.int32),
        jax.ShapeDtypeStruct((T, 1), jnp.int32),
        jax.ShapeDtypeStruct((T, 1), jnp.float32),
        jax.ShapeDtypeStruct((T, 1), jnp.float32),
        jax.ShapeDtypeStruct((NB, 1), jnp.int32),
        jax.ShapeDtypeStruct((1, 1), jnp.int32),
    )
    return pl.pallas_call(_router_body, out_shape=out_shapes)(x2d, gate_w)


def _mm_body(be_ref, used_ref, x_ref, wg_ref, wu_ref, wd_ref, rw_ref, out_ref):
    b = pl.program_id(0)

    @pl.when(b < used_ref[0])
    def _():
        x = x_ref[...].astype(jnp.bfloat16)
        g = jnp.dot(x, wg_ref[0].astype(jnp.bfloat16),
                    preferred_element_type=jnp.float32)
        u = jnp.dot(x, wu_ref[0].astype(jnp.bfloat16),
                    preferred_element_type=jnp.float32)
        act = ((g * jax.nn.sigmoid(g)) * u).astype(jnp.bfloat16)
        y = jnp.dot(act, wd_ref[0].astype(jnp.bfloat16),
                    preferred_element_type=jnp.float32)
        out_ref[...] = y * rw_ref[...]


def _run_grouped_mm(xs, w_gate, w_up, w_down, rw_sorted, be, used):
    def live(b, be_ref, used_ref):
        return (jnp.minimum(b, used_ref[0] - 1), 0)

    grid_spec = pltpu.PrefetchScalarGridSpec(
        num_scalar_prefetch=2,
        grid=(NB,),
        in_specs=[
            pl.BlockSpec((BM, D), live),
            pl.BlockSpec((1, D, F), lambda b, be_ref, u: (be_ref[b], 0, 0)),
            pl.BlockSpec((1, D, F), lambda b, be_ref, u: (be_ref[b], 0, 0)),
            pl.BlockSpec((1, F, D), lambda b, be_ref, u: (be_ref[b], 0, 0)),
            pl.BlockSpec((BM, 1), live),
        ],
        out_specs=pl.BlockSpec((BM, D), live),
    )
    return pl.pallas_call(
        _mm_body,
        grid_spec=grid_spec,
        out_shape=jax.ShapeDtypeStruct((P, D), jnp.float32),
    )(be, used, xs, w_gate, w_up, w_down, rw_sorted)


def kernel(hidden_states, gate_w, w_gate, w_up, w_down):
    b, s, d = hidden_states.shape
    x2d = hidden_states.reshape(-1, d)

    d0, d1, w1, w2, be, used = _run_router(x2d, gate_w)

    dest = jnp.concatenate([d0, d1], axis=1).reshape(-1)
    rwflat = jnp.concatenate([w1, w2], axis=1).reshape(-1)
    src_idx = jnp.zeros(P, jnp.int32).at[dest].set(
        (jnp.arange(T * K) // K).astype(jnp.int32))
    rw_sorted = jnp.zeros((P, 1), jnp.float32).at[dest, 0].set(rwflat)

    xs = jnp.take(x2d, src_idx, axis=0)

    y = _run_grouped_mm(xs, w_gate, w_up, w_down, rw_sorted,
                        be.reshape(-1), used.reshape(-1))

    out = jnp.take(y, d0[:, 0], axis=0) + jnp.take(y, d1[:, 0], axis=0)
    return out.reshape(b, s, d)


# speedup vs baseline: 1.0784x; 1.0784x over previous
import functools

import jax
import jax.numpy as jnp
from jax.experimental import pallas as pl
from jax.experimental.pallas import tpu as pltpu

T = 2048
D = 1024
E = 64
F = 512
K = 2
BM = 128
P = 12288
NB = P // BM


CH = 128
NCH = T // CH


def _router_body(x_ref, gw_ref, d0_ref, d1_ref, w1_ref, w2_ref,
                 be_ref, used_ref):
    x = x_ref[...]
    gw = gw_ref[...]
    logits = jax.lax.dot_general(x, gw, (((1,), (1,)), ((), ())),
                                 preferred_element_type=jnp.float32)
    m1 = jnp.max(logits, axis=1, keepdims=True)
    iota = jax.lax.broadcasted_iota(jnp.int32, logits.shape, 1)
    big = jnp.int32(1 << 30)
    i1 = jnp.min(jnp.where(logits == m1, iota, big), axis=1, keepdims=True)
    mask1 = iota == i1
    m2 = jnp.max(jnp.where(mask1, -jnp.inf, logits), axis=1, keepdims=True)
    i2 = jnp.min(jnp.where((logits == m2) & (~mask1), iota, big),
                 axis=1, keepdims=True)
    w1 = 1.0 / (1.0 + jnp.exp(m2 - m1))
    w1_ref[...] = w1
    w2_ref[...] = 1.0 - w1

    oh1 = (iota == i1).astype(jnp.float32)
    oh2 = (iota == i2).astype(jnp.float32)
    ohc = oh1 + oh2
    ri = jax.lax.broadcasted_iota(jnp.int32, (CH, CH), 0)
    ci = jax.lax.broadcasted_iota(jnp.int32, (CH, CH), 1)
    tril = (ci < ri).astype(jnp.float32)
    base_rows = []
    running = jnp.zeros((1, E), jnp.float32)
    for c in range(NCH):
        chunk = ohc[c * CH:(c + 1) * CH]
        excl = jnp.dot(tril, chunk, preferred_element_type=jnp.float32)
        base_rows.append(excl + running)
        running = running + jnp.sum(chunk, axis=0, keepdims=True)
    base = jnp.concatenate(base_rows, axis=0)
    counts = running

    nblk = jnp.ceil(counts / BM)
    ei = jax.lax.broadcasted_iota(jnp.int32, (E, E), 0)
    ej = jax.lax.broadcasted_iota(jnp.int32, (E, E), 1)
    triu_inc = (ei <= ej).astype(jnp.float32)
    bend = jnp.dot(nblk, triu_inc, preferred_element_type=jnp.float32)
    pstart = (bend - nblk) * BM
    d0 = jnp.sum(oh1 * (pstart + base), axis=1, keepdims=True)
    d1 = jnp.sum(oh2 * (pstart + base), axis=1, keepdims=True)
    d0_ref[...] = d0.astype(jnp.int32)
    d1_ref[...] = d1.astype(jnp.int32)

    used2d = jax.lax.slice(bend, (0, E - 1), (1, E))
    used = used2d[0, 0]
    used_ref[...] = used2d.astype(jnp.int32)
    blki = jax.lax.broadcasted_iota(jnp.int32, (NB, E), 0).astype(jnp.float32)
    be_raw = jnp.sum((bend <= blki).astype(jnp.float32), axis=1, keepdims=True)
    eidx = jax.lax.broadcasted_iota(jnp.int32, (1, E), 1).astype(jnp.float32)
    last_e = jnp.max(jnp.where(counts > 0, eidx, -1.0))
    blk1 = jax.lax.broadcasted_iota(jnp.int32, (NB, 1), 0).astype(jnp.float32)
    be = jnp.where(blk1 < used, be_raw, last_e)
    be_ref[...] = be.astype(jnp.int32)


def _run_router(x2d, gate_w):
    out_shapes = (
        jax.ShapeDtypeStruct((T, 1), jnp.int32),
        jax.ShapeDtypeStruct((T, 1), jnp.int32),
        jax.ShapeDtypeStruct((T, 1), jnp.float32),
        jax.ShapeDtypeStruct((T, 1), jnp.float32),
        jax.ShapeDtypeStruct((NB, 1), jnp.int32),
        jax.ShapeDtypeStruct((1, 1), jnp.int32),
    )
    return pl.pallas_call(_router_body, out_shape=out_shapes)(x2d, gate_w)


def _mm_body(be_ref, used_ref, x_ref, wg_ref, wu_ref, wd_ref, rw_ref, out_ref):
    b = pl.program_id(0)

    @pl.when(b < used_ref[0])
    def _():
        x = x_ref[...].astype(jnp.bfloat16)
        g = jnp.dot(x, wg_ref[0].astype(jnp.bfloat16),
                    preferred_element_type=jnp.float32)
        u = jnp.dot(x, wu_ref[0].astype(jnp.bfloat16),
                    preferred_element_type=jnp.float32)
        act = ((g * jax.nn.sigmoid(g)) * u).astype(jnp.bfloat16)
        y = jnp.dot(act, wd_ref[0].astype(jnp.bfloat16),
                    preferred_element_type=jnp.float32)
        out_ref[...] = y * rw_ref[...]


def _run_grouped_mm(xs, w_gate, w_up, w_down, rw_sorted, be, used):
    def live(b, be_ref, used_ref):
        return (jnp.minimum(b, used_ref[0] - 1), 0)

    grid_spec = pltpu.PrefetchScalarGridSpec(
        num_scalar_prefetch=2,
        grid=(NB,),
        in_specs=[
            pl.BlockSpec((BM, D), live),
            pl.BlockSpec((1, D, F), lambda b, be_ref, u: (be_ref[b], 0, 0)),
            pl.BlockSpec((1, D, F), lambda b, be_ref, u: (be_ref[b], 0, 0)),
            pl.BlockSpec((1, F, D), lambda b, be_ref, u: (be_ref[b], 0, 0)),
            pl.BlockSpec((BM, 1), live),
        ],
        out_specs=pl.BlockSpec((BM, D), live),
    )
    return pl.pallas_call(
        _mm_body,
        grid_spec=grid_spec,
        out_shape=jax.ShapeDtypeStruct((P, D), jnp.float32),
    )(be, used, xs, w_gate, w_up, w_down, rw_sorted)


def kernel(hidden_states, gate_w, w_gate, w_up, w_down):
    b, s, d = hidden_states.shape
    x2d = hidden_states.reshape(-1, d)

    d0, d1, w1, w2, be, used = _run_router(x2d, gate_w)

    dest = jnp.concatenate([d0, d1], axis=1).reshape(-1)
    rwflat = jnp.concatenate([w1, w2], axis=1).reshape(-1)
    src_idx = jnp.zeros(P, jnp.int32).at[dest].set(
        (jnp.arange(T * K) // K).astype(jnp.int32))
    rw_sorted = jnp.zeros((P, 1), jnp.float32).at[dest, 0].set(rwflat)

    xs = jnp.take(x2d, src_idx, axis=0)

    y = _run_grouped_mm(xs, w_gate, w_up, w_down, rw_sorted,
                        be.reshape(-1), used.reshape(-1))

    out = jnp.take(y, d0[:, 0], axis=0) + jnp.take(y, d1[:, 0], axis=0)
    return out.reshape(b, s, d)
